# baseline (speedup 1.0000x reference)
import functools

import jax
import jax.numpy as jnp
from jax import lax
from jax.experimental import pallas as pl
from jax.experimental.pallas import tpu as pltpu
from jax.experimental.pallas import tpu_sc as plsc

_L = 16
_UNROLL = 8
_CHUNK = 12800


def _sc_convert(species_flat, conv_tensor):
    n = species_flat.shape[0]
    info = plsc.get_sparse_core_info()
    nc, ns = info.num_cores, info.num_subcores
    nw = nc * ns
    per_w = n // nw
    n_chunks = per_w // _CHUNK
    assert per_w * nw == n and n_chunks * _CHUNK == per_w and n_chunks % 2 == 0
    table_n = conv_tensor.shape[0]

    mesh = plsc.VectorSubcoreMesh(core_axis_name="c", subcore_axis_name="s")

    @functools.partial(
        pl.kernel,
        mesh=mesh,
        compiler_params=pltpu.CompilerParams(needs_layout_passes=False),
        out_type=jax.ShapeDtypeStruct((n,), jnp.int32),
        scratch_types=[
            pltpu.VMEM((table_n,), jnp.int32),
            pltpu.VMEM((_CHUNK,), jnp.int32),
            pltpu.VMEM((_CHUNK,), jnp.int32),
            pltpu.VMEM((_CHUNK,), jnp.int32),
            pltpu.VMEM((_CHUNK,), jnp.int32),
            pltpu.SemaphoreType.DMA,
            pltpu.SemaphoreType.DMA,
            pltpu.SemaphoreType.DMA,
            pltpu.SemaphoreType.DMA,
        ],
    )
    def k(species_hbm, conv_hbm, out_hbm, table_v, in0, in1, out0, out1,
          s_in0, s_in1, s_out0, s_out1):
        wid = lax.axis_index("s") * nc + lax.axis_index("c")
        pltpu.sync_copy(conv_hbm, table_v)
        base0 = wid * per_w

        def in_slice(ci):
            return species_hbm.at[pl.ds(base0 + ci * _CHUNK, _CHUNK)]

        def out_slice(ci):
            return out_hbm.at[pl.ds(base0 + ci * _CHUNK, _CHUNK)]

        def convert(in_v, out_v):
            def body(j, c):
                b = j * (_UNROLL * _L)
                for u in range(_UNROLL):
                    off = b + u * _L
                    idx = in_v[pl.ds(off, _L)]
                    out_v[pl.ds(off, _L)] = plsc.load_gather(table_v, [idx])
                return c

            lax.fori_loop(0, _CHUNK // (_UNROLL * _L), body, 0)

        pltpu.async_copy(in_slice(0), in0, s_in0)
        pltpu.async_copy(in_slice(1), in1, s_in1)

        def round_body(i, c):
            c0 = 2 * i
            c1 = c0 + 1

            pltpu.make_async_copy(in_slice(c0), in0, s_in0).wait()

            @pl.when(i > 0)
            def _():
                pltpu.make_async_copy(out0, out_slice(c0), s_out0).wait()

            convert(in0, out0)
            pltpu.async_copy(out0, out_slice(c0), s_out0)

            @pl.when(c0 + 2 < n_chunks)
            def _():
                pltpu.async_copy(in_slice(c0 + 2), in0, s_in0)

            pltpu.make_async_copy(in_slice(c1), in1, s_in1).wait()

            @pl.when(i > 0)
            def _():
                pltpu.make_async_copy(out1, out_slice(c1), s_out1).wait()

            convert(in1, out1)
            pltpu.async_copy(out1, out_slice(c1), s_out1)

            @pl.when(c1 + 2 < n_chunks)
            def _():
                pltpu.async_copy(in_slice(c1 + 2), in1, s_in1)

            return c

        lax.fori_loop(0, n_chunks // 2, round_body, 0)

        pltpu.make_async_copy(out0, out_slice(n_chunks - 2), s_out0).wait()
        pltpu.make_async_copy(out1, out_slice(n_chunks - 1), s_out1).wait()

    return k(species_flat, conv_tensor)


def _tc_copy(x2d):
    rows, cols = x2d.shape
    block_rows = 512
    grid = rows // block_rows

    def body(x_ref, o_ref):
        o_ref[...] = x_ref[...]

    return pl.pallas_call(
        body,
        grid=(grid,),
        in_specs=[pl.BlockSpec((block_rows, cols), lambda i: (i, 0))],
        out_specs=pl.BlockSpec((block_rows, cols), lambda i: (i, 0)),
        out_shape=jax.ShapeDtypeStruct((rows, cols), x2d.dtype),
    )(x2d)


def kernel(species, coordinates, conv_tensor):
    converted = _sc_convert(species.reshape(-1), conv_tensor)
    b, a, c = coordinates.shape
    coords_out = _tc_copy(coordinates.reshape(b, a * c)).reshape(b, a, c)
    return converted.reshape(species.shape), coords_out


# speedup vs baseline: 4.2720x; 4.2720x over previous
import functools

import jax
import jax.numpy as jnp
from jax import lax
from jax.experimental import pallas as pl
from jax.experimental.pallas import tpu as pltpu
from jax.experimental.pallas import tpu_sc as plsc

_L = 16
_UNROLL = 8
_CHUNK = 12800


def _sc_convert(species_flat, conv_tensor):
    n = species_flat.shape[0]
    info = plsc.get_sparse_core_info()
    nc, ns = info.num_cores, info.num_subcores
    nw = nc * ns
    per_w = n // nw
    n_chunks = per_w // _CHUNK
    assert per_w * nw == n and n_chunks * _CHUNK == per_w and n_chunks % 2 == 0
    table_n = conv_tensor.shape[0]

    mesh = plsc.VectorSubcoreMesh(core_axis_name="c", subcore_axis_name="s")

    @functools.partial(
        pl.kernel,
        mesh=mesh,
        compiler_params=pltpu.CompilerParams(needs_layout_passes=False),
        out_type=jax.ShapeDtypeStruct((n,), jnp.int32),
        scratch_types=[
            pltpu.VMEM((table_n,), jnp.int32),
            pltpu.VMEM((_CHUNK,), jnp.int32),
            pltpu.VMEM((_CHUNK,), jnp.int32),
            pltpu.VMEM((_CHUNK,), jnp.int32),
            pltpu.VMEM((_CHUNK,), jnp.int32),
            pltpu.SemaphoreType.DMA,
            pltpu.SemaphoreType.DMA,
            pltpu.SemaphoreType.DMA,
            pltpu.SemaphoreType.DMA,
        ],
    )
    def k(species_hbm, conv_hbm, out_hbm, table_v, in0, in1, out0, out1,
          s_in0, s_in1, s_out0, s_out1):
        wid = lax.axis_index("s") * nc + lax.axis_index("c")
        pltpu.sync_copy(conv_hbm, table_v)
        base0 = wid * per_w

        def in_slice(ci):
            return species_hbm.at[pl.ds(base0 + ci * _CHUNK, _CHUNK)]

        def out_slice(ci):
            return out_hbm.at[pl.ds(base0 + ci * _CHUNK, _CHUNK)]

        def convert(in_v, out_v):
            def body(j, c):
                b = j * (_UNROLL * _L)
                for u in range(_UNROLL):
                    off = b + u * _L
                    idx = in_v[pl.ds(off, _L)]
                    out_v[pl.ds(off, _L)] = plsc.load_gather(table_v, [idx])
                return c

            lax.fori_loop(0, _CHUNK // (_UNROLL * _L), body, 0)

        pltpu.async_copy(in_slice(0), in0, s_in0)
        pltpu.async_copy(in_slice(1), in1, s_in1)

        def round_body(i, c):
            c0 = 2 * i
            c1 = c0 + 1

            pltpu.make_async_copy(in_slice(c0), in0, s_in0).wait()

            @pl.when(i > 0)
            def _():
                pltpu.make_async_copy(out0, out_slice(c0), s_out0).wait()

            convert(in0, out0)
            pltpu.async_copy(out0, out_slice(c0), s_out0)

            @pl.when(c0 + 2 < n_chunks)
            def _():
                pltpu.async_copy(in_slice(c0 + 2), in0, s_in0)

            pltpu.make_async_copy(in_slice(c1), in1, s_in1).wait()

            @pl.when(i > 0)
            def _():
                pltpu.make_async_copy(out1, out_slice(c1), s_out1).wait()

            convert(in1, out1)
            pltpu.async_copy(out1, out_slice(c1), s_out1)

            @pl.when(c1 + 2 < n_chunks)
            def _():
                pltpu.async_copy(in_slice(c1 + 2), in1, s_in1)

            return c

        lax.fori_loop(0, n_chunks // 2, round_body, 0)

        pltpu.make_async_copy(out0, out_slice(n_chunks - 2), s_out0).wait()
        pltpu.make_async_copy(out1, out_slice(n_chunks - 1), s_out1).wait()

    return k(species_flat, conv_tensor)


def kernel(species, coordinates, conv_tensor):
    converted = _sc_convert(species.reshape(-1), conv_tensor)
    return converted.reshape(species.shape), coordinates
